# baseline (speedup 1.0000x reference)
import jax
import jax.numpy as jnp
from jax import lax
from jax.experimental import pallas as pl
from jax.experimental.pallas import tpu as pltpu
from jax.experimental.pallas import tpu_sc as plsc

NC = 2
NS = 16
NW = NC * NS
CB = 128
H = 32
CNTW = 16


def _sc_mesh():
    return plsc.VectorSubcoreMesh(
        core_axis_name="c", subcore_axis_name="s", num_cores=NC, num_subcores=NS
    )


def _zero_fill(ref, nrows, ncols):
    z = jnp.zeros((16,), jnp.float32)

    def row(i, _):
        for c0 in range(0, ncols, 16):
            ref[i, pl.ds(c0, 16)] = z
        return 0

    lax.fori_loop(0, nrows, row, 0)


def _one_fill(ref, nrows, ncols):
    o = jnp.ones((16,), jnp.float32)

    def row(i, _):
        for c0 in range(0, ncols, 16):
            ref[i, pl.ds(c0, 16)] = o
        return 0

    lax.fori_loop(0, nrows, row, 0)


def _make_sc_pass(n, n_acc, ch, with_cnt, nb):
    rpt = n_acc // NS
    spt = n // NS
    assert n % NS == 0 and ch % nb == 0

    out_type = [jax.ShapeDtypeStruct((NC, n_acc, H), jnp.float32)]
    scratch = [
        pltpu.VMEM((ch, CB), jnp.int32),
        pltpu.VMEM((ch, CB), jnp.int32),
        [pltpu.VMEM((CB, H), jnp.float32) for _ in range(nb)],
        pltpu.VMEM((rpt, H), jnp.float32),
        pltpu.VMEM_SHARED((n, H), jnp.float32),
        pltpu.VMEM_SHARED((n_acc, H), jnp.float32),
        [pltpu.SemaphoreType.DMA for _ in range(nb)],
        [pltpu.SemaphoreType.DMA for _ in range(nb)],
    ]
    if with_cnt:
        out_type.append(jax.ShapeDtypeStruct((NC, n_acc, CNTW), jnp.float32))
        scratch += [
            pltpu.VMEM((CB, CNTW), jnp.float32),
            pltpu.VMEM((rpt, CNTW), jnp.float32),
            pltpu.VMEM_SHARED((n_acc, CNTW), jnp.float32),
        ]

    def body(p_hbm, src_hbm, dst_hbm, out_hbm, *rest):
        if with_cnt:
            (cnt_hbm, idx_s, idx_d, rows, zbuf, psh, acc, sem_g, sem_s,
             ones, zcnt, acc_cnt) = rest
        else:
            idx_s, idx_d, rows, zbuf, psh, acc, sem_g, sem_s = rest
        c = lax.axis_index("c")
        s = lax.axis_index("s")
        wid = c * NS + s

        pltpu.async_copy(p_hbm.at[pl.ds(s * spt, spt)],
                         psh.at[pl.ds(s * spt, spt)], sem_g[0])
        _zero_fill(zbuf, rpt, H)
        pltpu.sync_copy(zbuf, acc.at[pl.ds(s * rpt, rpt)])
        if with_cnt:
            _zero_fill(zcnt, rpt, CNTW)
            pltpu.sync_copy(zcnt, acc_cnt.at[pl.ds(s * rpt, rpt)])
            _one_fill(ones, CB, CNTW)
        pltpu.sync_copy(src_hbm.at[wid], idx_s)
        pltpu.sync_copy(dst_hbm.at[wid], idx_d)
        pltpu.make_async_copy(p_hbm.at[pl.ds(s * spt, spt)],
                              psh.at[pl.ds(s * spt, spt)], sem_g[0]).wait()
        plsc.subcore_barrier()

        def group(g, _):
            j0 = g * nb
            gat = [
                pltpu.async_copy(psh.at[idx_s.at[j0 + b]], rows[b], sem_g[b])
                for b in range(nb)
            ]
            sca = []
            for b in range(nb):
                gat[b].wait()
                sca.append(pltpu.async_copy(
                    rows[b], acc.at[idx_d.at[j0 + b]], sem_s[b], add=True))
                if with_cnt:
                    sca.append(pltpu.async_copy(
                        ones, acc_cnt.at[idx_d.at[j0 + b]], sem_s[b], add=True))
            for d in sca:
                d.wait()
            return 0

        lax.fori_loop(0, ch // nb, group, 0)
        plsc.subcore_barrier()

        pltpu.sync_copy(acc.at[pl.ds(s * rpt, rpt)], zbuf)
        pltpu.sync_copy(zbuf, out_hbm.at[c, pl.ds(s * rpt, rpt)])
        if with_cnt:
            pltpu.sync_copy(acc_cnt.at[pl.ds(s * rpt, rpt)], zcnt)
            pltpu.sync_copy(zcnt, cnt_hbm.at[c, pl.ds(s * rpt, rpt)])

    return pl.kernel(
        body, out_type=out_type, mesh=_sc_mesh(), scratch_types=scratch,
        compiler_params=pltpu.CompilerParams(use_tc_tiling_on_sc=False),
    )


def _tc0_body(x_ref, wl_ref, wr_ref, bl_ref, p_ref, r_ref):
    x = x_ref[...]
    p_ref[...] = jnp.dot(x, wl_ref[...], preferred_element_type=jnp.float32)
    r_ref[...] = (
        jnp.dot(x, wr_ref[...], preferred_element_type=jnp.float32) + bl_ref[...]
    )


def _bn_relu(out, g, b):
    mu = jnp.mean(out, axis=0, keepdims=True)
    d = out - mu
    var = jnp.mean(d * d, axis=0, keepdims=True)
    return jnp.maximum(d * lax.rsqrt(var + 1e-5) * g + b, 0.0)


def _make_tc_first(n):
    def body(part_ref, cntp_ref, r_ref, g_ref, b_ref, wl_ref, wr_ref, bl_ref,
             cinv_ref, p_ref, r2_ref):
        cnt = cntp_ref[0, :n, 0:1] + cntp_ref[1, :n, 0:1]
        cinv = 1.0 / jnp.maximum(cnt, 1.0)
        cinv_ref[...] = cinv
        agg = (part_ref[0, :n, :] + part_ref[1, :n, :]) * cinv
        h = _bn_relu(agg + r_ref[...], g_ref[...], b_ref[...])
        p_ref[...] = jnp.dot(h, wl_ref[...], preferred_element_type=jnp.float32)
        r2_ref[...] = (
            jnp.dot(h, wr_ref[...], preferred_element_type=jnp.float32) + bl_ref[...]
        )

    return body


def _make_tc_mid(n, project):
    def body(part_ref, cinv_ref, r_ref, g_ref, b_ref, wl_ref, wr_ref, bl_ref,
             p_ref, r2_ref):
        agg = (part_ref[0, :n, :] + part_ref[1, :n, :]) * cinv_ref[...]
        h = _bn_relu(agg + r_ref[...], g_ref[...], b_ref[...])
        if project:
            p_ref[...] = jnp.dot(h, wl_ref[...], preferred_element_type=jnp.float32)
        else:
            p_ref[...] = h
        r2_ref[...] = (
            jnp.dot(h, wr_ref[...], preferred_element_type=jnp.float32) + bl_ref[...]
        )

    return body


def _make_tc_final(n, ng):
    def body(part_ref, cinv_ref, r_ref, wl_ref, batch_ref, out_ref):
        agg = (part_ref[0, :n, :] + part_ref[1, :n, :]) * cinv_ref[...]
        out = (
            jnp.dot(agg, wl_ref[...], preferred_element_type=jnp.float32) + r_ref[...]
        )
        gid = lax.broadcasted_iota(jnp.int32, (1, ng), 1)
        onehot = (batch_ref[...] == gid).astype(jnp.float32)
        s = lax.dot_general(
            onehot, out, (((0,), (0,)), ((), ())),
            preferred_element_type=jnp.float32,
        )
        gc = lax.dot_general(
            onehot, jnp.ones((n, 1), jnp.float32), (((0,), (0,)), ((), ())),
            preferred_element_type=jnp.float32,
        )
        out_ref[...] = s / jnp.maximum(gc, 1.0)

    return body


def kernel(x, edge_index, batch, Wl0, bl0, Wr0, Wl_mid, bl_mid, Wr_mid,
           Wl_last, bl_last, Wr_last, gamma, beta):
    n, d = x.shape
    e = edge_index.shape[1]
    ng = 64
    out_dim = Wl_last.shape[1]

    ch = -(-e // (NW * CB))
    ch = -(-ch // 8) * 8
    e_pad = NW * ch * CB
    n_acc = -(-(n + 1) // (NS * 8)) * (NS * 8)

    src = edge_index[0]
    dst = edge_index[1]
    pad = e_pad - e
    src_p = jnp.concatenate([src, jnp.zeros((pad,), jnp.int32)]).reshape(NW, ch, CB)
    dst_p = jnp.concatenate([dst, jnp.full((pad,), n, jnp.int32)]).reshape(NW, ch, CB)

    sc_first = _make_sc_pass(n, n_acc, ch, with_cnt=True, nb=4)
    sc_rest = _make_sc_pass(n, n_acc, ch, with_cnt=False, nb=8)

    f32 = jnp.float32
    tc0 = pl.pallas_call(
        _tc0_body,
        out_shape=[jax.ShapeDtypeStruct((n, H), f32)] * 2,
    )
    tc_first = pl.pallas_call(
        _make_tc_first(n),
        out_shape=[
            jax.ShapeDtypeStruct((n, 1), f32),
            jax.ShapeDtypeStruct((n, H), f32),
            jax.ShapeDtypeStruct((n, H), f32),
        ],
    )
    tc_mid = pl.pallas_call(
        _make_tc_mid(n, True),
        out_shape=[jax.ShapeDtypeStruct((n, H), f32)] * 2,
    )
    tc_pre_last = pl.pallas_call(
        _make_tc_mid(n, False),
        out_shape=[
            jax.ShapeDtypeStruct((n, H), f32),
            jax.ShapeDtypeStruct((n, out_dim), f32),
        ],
    )
    tc_final = pl.pallas_call(
        _make_tc_final(n, ng),
        out_shape=jax.ShapeDtypeStruct((ng, out_dim), f32),
    )

    g2 = gamma.reshape(5, 1, H)
    b2 = beta.reshape(5, 1, H)

    p, r = tc0(x, Wl0, Wr0, bl0.reshape(1, H))
    part, cntp = sc_first(p, src_p, dst_p)
    cinv, p, r = tc_first(
        part, cntp, r, g2[0], b2[0],
        Wl_mid[0], Wr_mid[0], bl_mid[0].reshape(1, H),
    )
    for i in range(1, 4):
        part, = sc_rest(p, src_p, dst_p)
        p, r = tc_mid(
            part, cinv, r, g2[i], b2[i],
            Wl_mid[i], Wr_mid[i], bl_mid[i].reshape(1, H),
        )
    part, = sc_rest(p, src_p, dst_p)
    p, r = tc_pre_last(
        part, cinv, r, g2[4], b2[4],
        Wl_mid[3], Wr_last, bl_last.reshape(1, out_dim),
    )
    part, = sc_rest(p, src_p, dst_p)
    return tc_final(part, cinv, r, Wl_last, batch.reshape(n, 1))


# speedup vs baseline: 1.0804x; 1.0804x over previous
import jax
import jax.numpy as jnp
from jax import lax
from jax.experimental import pallas as pl
from jax.experimental.pallas import tpu as pltpu
from jax.experimental.pallas import tpu_sc as plsc

NC = 2
NS = 16
NW = NC * NS
CB = 128
H = 32
CNTW = 16


def _sc_mesh():
    return plsc.VectorSubcoreMesh(
        core_axis_name="c", subcore_axis_name="s", num_cores=NC, num_subcores=NS
    )


def _zero_fill(ref, nrows, ncols):
    z = jnp.zeros((16,), jnp.float32)

    def row(i, _):
        for c0 in range(0, ncols, 16):
            ref[i, pl.ds(c0, 16)] = z
        return 0

    lax.fori_loop(0, nrows, row, 0)


def _one_fill(ref, nrows, ncols):
    o = jnp.ones((16,), jnp.float32)

    def row(i, _):
        for c0 in range(0, ncols, 16):
            ref[i, pl.ds(c0, 16)] = o
        return 0

    lax.fori_loop(0, nrows, row, 0)


def _make_sc_pass(n, n_acc, ch, with_cnt, nb):
    rpt = n_acc // NS
    spt = n // NS
    assert n % NS == 0 and ch % nb == 0

    out_type = [jax.ShapeDtypeStruct((NC, n_acc, H), jnp.float32)]
    scratch = [
        pltpu.VMEM((ch, CB), jnp.int32),
        pltpu.VMEM((ch, CB), jnp.int32),
        [pltpu.VMEM((CB, H), jnp.float32) for _ in range(nb)],
        pltpu.VMEM((rpt, H), jnp.float32),
        pltpu.VMEM_SHARED((n, H), jnp.float32),
        pltpu.VMEM_SHARED((n_acc, H), jnp.float32),
        pltpu.SemaphoreType.DMA,
        pltpu.SemaphoreType.DMA,
    ]
    if with_cnt:
        out_type.append(jax.ShapeDtypeStruct((NC, n_acc, CNTW), jnp.float32))
        scratch += [
            pltpu.VMEM((CB, CNTW), jnp.float32),
            pltpu.VMEM((rpt, CNTW), jnp.float32),
            pltpu.VMEM_SHARED((n_acc, CNTW), jnp.float32),
        ]

    def body(p_hbm, src_hbm, dst_hbm, out_hbm, *rest):
        if with_cnt:
            (cnt_hbm, idx_s, idx_d, rows, zbuf, psh, acc, sem_g, sem_s,
             ones, zcnt, acc_cnt) = rest
        else:
            idx_s, idx_d, rows, zbuf, psh, acc, sem_g, sem_s = rest
        c = lax.axis_index("c")
        s = lax.axis_index("s")
        wid = c * NS + s

        pltpu.async_copy(p_hbm.at[pl.ds(s * spt, spt)],
                         psh.at[pl.ds(s * spt, spt)], sem_g)
        _zero_fill(zbuf, rpt, H)
        pltpu.sync_copy(zbuf, acc.at[pl.ds(s * rpt, rpt)])
        if with_cnt:
            _zero_fill(zcnt, rpt, CNTW)
            pltpu.sync_copy(zcnt, acc_cnt.at[pl.ds(s * rpt, rpt)])
            _one_fill(ones, CB, CNTW)
        pltpu.sync_copy(src_hbm.at[wid], idx_s)
        pltpu.sync_copy(dst_hbm.at[wid], idx_d)
        pltpu.make_async_copy(p_hbm.at[pl.ds(s * spt, spt)],
                              psh.at[pl.ds(s * spt, spt)], sem_g).wait()
        plsc.subcore_barrier()

        def group(g, _):
            j0 = g * nb
            gat = [
                pltpu.async_copy(psh.at[idx_s.at[j0 + b]], rows[b], sem_g)
                for b in range(nb)
            ]
            sca = []
            for b in range(nb):
                gat[b].wait()
                sca.append(pltpu.async_copy(
                    rows[b], acc.at[idx_d.at[j0 + b]], sem_s, add=True))
                if with_cnt:
                    sca.append(pltpu.async_copy(
                        ones, acc_cnt.at[idx_d.at[j0 + b]], sem_s, add=True))
            for d in sca:
                d.wait()
            return 0

        lax.fori_loop(0, ch // nb, group, 0)
        plsc.subcore_barrier()

        pltpu.sync_copy(acc.at[pl.ds(s * rpt, rpt)], zbuf)
        pltpu.sync_copy(zbuf, out_hbm.at[c, pl.ds(s * rpt, rpt)])
        if with_cnt:
            pltpu.sync_copy(acc_cnt.at[pl.ds(s * rpt, rpt)], zcnt)
            pltpu.sync_copy(zcnt, cnt_hbm.at[c, pl.ds(s * rpt, rpt)])

    return pl.kernel(
        body, out_type=out_type, mesh=_sc_mesh(), scratch_types=scratch,
        compiler_params=pltpu.CompilerParams(use_tc_tiling_on_sc=False),
    )


def _tc0_body(x_ref, wl_ref, wr_ref, bl_ref, p_ref, r_ref):
    x = x_ref[...]
    p_ref[...] = jnp.dot(x, wl_ref[...], preferred_element_type=jnp.float32)
    r_ref[...] = (
        jnp.dot(x, wr_ref[...], preferred_element_type=jnp.float32) + bl_ref[...]
    )


def _bn_relu(out, g, b):
    mu = jnp.mean(out, axis=0, keepdims=True)
    d = out - mu
    var = jnp.mean(d * d, axis=0, keepdims=True)
    return jnp.maximum(d * lax.rsqrt(var + 1e-5) * g + b, 0.0)


def _make_tc_first(n):
    def body(part_ref, cntp_ref, r_ref, g_ref, b_ref, wl_ref, wr_ref, bl_ref,
             cinv_ref, p_ref, r2_ref):
        cnt = cntp_ref[0, :n, 0:1] + cntp_ref[1, :n, 0:1]
        cinv = 1.0 / jnp.maximum(cnt, 1.0)
        cinv_ref[...] = cinv
        agg = (part_ref[0, :n, :] + part_ref[1, :n, :]) * cinv
        h = _bn_relu(agg + r_ref[...], g_ref[...], b_ref[...])
        p_ref[...] = jnp.dot(h, wl_ref[...], preferred_element_type=jnp.float32)
        r2_ref[...] = (
            jnp.dot(h, wr_ref[...], preferred_element_type=jnp.float32) + bl_ref[...]
        )

    return body


def _make_tc_mid(n, project):
    def body(part_ref, cinv_ref, r_ref, g_ref, b_ref, wl_ref, wr_ref, bl_ref,
             p_ref, r2_ref):
        agg = (part_ref[0, :n, :] + part_ref[1, :n, :]) * cinv_ref[...]
        h = _bn_relu(agg + r_ref[...], g_ref[...], b_ref[...])
        if project:
            p_ref[...] = jnp.dot(h, wl_ref[...], preferred_element_type=jnp.float32)
        else:
            p_ref[...] = h
        r2_ref[...] = (
            jnp.dot(h, wr_ref[...], preferred_element_type=jnp.float32) + bl_ref[...]
        )

    return body


def _make_tc_final(n, ng):
    def body(part_ref, cinv_ref, r_ref, wl_ref, batch_ref, out_ref):
        agg = (part_ref[0, :n, :] + part_ref[1, :n, :]) * cinv_ref[...]
        out = (
            jnp.dot(agg, wl_ref[...], preferred_element_type=jnp.float32) + r_ref[...]
        )
        gid = lax.broadcasted_iota(jnp.int32, (1, ng), 1)
        onehot = (batch_ref[...] == gid).astype(jnp.float32)
        s = lax.dot_general(
            onehot, out, (((0,), (0,)), ((), ())),
            preferred_element_type=jnp.float32,
        )
        gc = lax.dot_general(
            onehot, jnp.ones((n, 1), jnp.float32), (((0,), (0,)), ((), ())),
            preferred_element_type=jnp.float32,
        )
        out_ref[...] = s / jnp.maximum(gc, 1.0)

    return body


def kernel(x, edge_index, batch, Wl0, bl0, Wr0, Wl_mid, bl_mid, Wr_mid,
           Wl_last, bl_last, Wr_last, gamma, beta):
    n, d = x.shape
    e = edge_index.shape[1]
    ng = 64
    out_dim = Wl_last.shape[1]

    ch = -(-e // (NW * CB))
    ch = -(-ch // 8) * 8
    e_pad = NW * ch * CB
    n_acc = -(-(n + 1) // (NS * 8)) * (NS * 8)

    src = edge_index[0]
    dst = edge_index[1]
    pad = e_pad - e
    src_p = jnp.concatenate([src, jnp.zeros((pad,), jnp.int32)]).reshape(NW, ch, CB)
    dst_p = jnp.concatenate([dst, jnp.full((pad,), n, jnp.int32)]).reshape(NW, ch, CB)

    sc_first = _make_sc_pass(n, n_acc, ch, with_cnt=True, nb=4)
    sc_rest = _make_sc_pass(n, n_acc, ch, with_cnt=False, nb=8)

    f32 = jnp.float32
    tc0 = pl.pallas_call(
        _tc0_body,
        out_shape=[jax.ShapeDtypeStruct((n, H), f32)] * 2,
    )
    tc_first = pl.pallas_call(
        _make_tc_first(n),
        out_shape=[
            jax.ShapeDtypeStruct((n, 1), f32),
            jax.ShapeDtypeStruct((n, H), f32),
            jax.ShapeDtypeStruct((n, H), f32),
        ],
    )
    tc_mid = pl.pallas_call(
        _make_tc_mid(n, True),
        out_shape=[jax.ShapeDtypeStruct((n, H), f32)] * 2,
    )
    tc_pre_last = pl.pallas_call(
        _make_tc_mid(n, False),
        out_shape=[
            jax.ShapeDtypeStruct((n, H), f32),
            jax.ShapeDtypeStruct((n, out_dim), f32),
        ],
    )
    tc_final = pl.pallas_call(
        _make_tc_final(n, ng),
        out_shape=jax.ShapeDtypeStruct((ng, out_dim), f32),
    )

    g2 = gamma.reshape(5, 1, H)
    b2 = beta.reshape(5, 1, H)

    p, r = tc0(x, Wl0, Wr0, bl0.reshape(1, H))
    part, cntp = sc_first(p, src_p, dst_p)
    cinv, p, r = tc_first(
        part, cntp, r, g2[0], b2[0],
        Wl_mid[0], Wr_mid[0], bl_mid[0].reshape(1, H),
    )
    for i in range(1, 4):
        part, = sc_rest(p, src_p, dst_p)
        p, r = tc_mid(
            part, cinv, r, g2[i], b2[i],
            Wl_mid[i], Wr_mid[i], bl_mid[i].reshape(1, H),
        )
    part, = sc_rest(p, src_p, dst_p)
    p, r = tc_pre_last(
        part, cinv, r, g2[4], b2[4],
        Wl_mid[3], Wr_last, bl_last.reshape(1, out_dim),
    )
    part, = sc_rest(p, src_p, dst_p)
    return tc_final(part, cinv, r, Wl_last, batch.reshape(n, 1))


# speedup vs baseline: 1.1129x; 1.0301x over previous
import jax
import jax.numpy as jnp
from jax import lax
from jax.experimental import pallas as pl
from jax.experimental.pallas import tpu as pltpu
from jax.experimental.pallas import tpu_sc as plsc

NC = 2
NS = 16
NW = NC * NS
CB = 128
H = 32
CNTW = 16


def _sc_mesh():
    return plsc.VectorSubcoreMesh(
        core_axis_name="c", subcore_axis_name="s", num_cores=NC, num_subcores=NS
    )


def _zero_fill(ref, nrows, ncols):
    z = jnp.zeros((16,), jnp.float32)

    def row(i, _):
        for c0 in range(0, ncols, 16):
            ref[i, pl.ds(c0, 16)] = z
        return 0

    lax.fori_loop(0, nrows, row, 0)


def _one_fill(ref, nrows, ncols):
    o = jnp.ones((16,), jnp.float32)

    def row(i, _):
        for c0 in range(0, ncols, 16):
            ref[i, pl.ds(c0, 16)] = o
        return 0

    lax.fori_loop(0, nrows, row, 0)


def _make_sc_pass(n, n_acc, ch, with_cnt, nb):
    rpt = n_acc // NS
    rpt2 = rpt // 2
    spt = n // NS
    assert n % NS == 0 and ch % nb == 0 and rpt % 2 == 0

    out_type = [jax.ShapeDtypeStruct((NC, n_acc, H), jnp.float32)]
    scratch = [
        pltpu.VMEM((ch, CB), jnp.int32),
        pltpu.VMEM((ch, CB), jnp.int32),
        [pltpu.VMEM((CB, H), jnp.float32) for _ in range(nb)],
        pltpu.VMEM((rpt2, H), jnp.float32),
        pltpu.VMEM_SHARED((n, H), jnp.float32),
        pltpu.VMEM_SHARED((n_acc, H), jnp.float32),
        pltpu.SemaphoreType.DMA,
        pltpu.SemaphoreType.DMA,
    ]
    if with_cnt:
        out_type.append(jax.ShapeDtypeStruct((NC, n_acc, CNTW), jnp.float32))
        scratch += [
            pltpu.VMEM((CB, CNTW), jnp.float32),
            pltpu.VMEM((rpt2, CNTW), jnp.float32),
            pltpu.VMEM_SHARED((n_acc, CNTW), jnp.float32),
        ]

    def body(p_hbm, src_hbm, dst_hbm, out_hbm, *rest):
        if with_cnt:
            (cnt_hbm, idx_s, idx_d, rows, zbuf, psh, acc, sem_g, sem_s,
             ones, zcnt, acc_cnt) = rest
        else:
            idx_s, idx_d, rows, zbuf, psh, acc, sem_g, sem_s = rest
        c = lax.axis_index("c")
        s = lax.axis_index("s")
        wid = c * NS + s

        pltpu.async_copy(p_hbm.at[pl.ds(s * spt, spt)],
                         psh.at[pl.ds(s * spt, spt)], sem_g)
        _zero_fill(zbuf, rpt2, H)
        for k in range(2):
            pltpu.sync_copy(zbuf, acc.at[pl.ds(s * rpt + k * rpt2, rpt2)])
        if with_cnt:
            _zero_fill(zcnt, rpt2, CNTW)
            for k in range(2):
                pltpu.sync_copy(
                    zcnt, acc_cnt.at[pl.ds(s * rpt + k * rpt2, rpt2)])
            _one_fill(ones, CB, CNTW)
        pltpu.sync_copy(src_hbm.at[wid], idx_s)
        pltpu.sync_copy(dst_hbm.at[wid], idx_d)
        pltpu.make_async_copy(p_hbm.at[pl.ds(s * spt, spt)],
                              psh.at[pl.ds(s * spt, spt)], sem_g).wait()
        plsc.subcore_barrier()

        def group(g, _):
            j0 = g * nb
            gat = [
                pltpu.async_copy(psh.at[idx_s.at[j0 + b]], rows[b], sem_g)
                for b in range(nb)
            ]
            sca = []
            for b in range(nb):
                gat[b].wait()
                sca.append(pltpu.async_copy(
                    rows[b], acc.at[idx_d.at[j0 + b]], sem_s, add=True))
                if with_cnt:
                    sca.append(pltpu.async_copy(
                        ones, acc_cnt.at[idx_d.at[j0 + b]], sem_s, add=True))
            for d in sca:
                d.wait()
            return 0

        lax.fori_loop(0, ch // nb, group, 0)
        plsc.subcore_barrier()

        for k in range(2):
            pltpu.sync_copy(acc.at[pl.ds(s * rpt + k * rpt2, rpt2)], zbuf)
            pltpu.sync_copy(zbuf, out_hbm.at[c, pl.ds(s * rpt + k * rpt2, rpt2)])
        if with_cnt:
            for k in range(2):
                pltpu.sync_copy(
                    acc_cnt.at[pl.ds(s * rpt + k * rpt2, rpt2)], zcnt)
                pltpu.sync_copy(
                    zcnt, cnt_hbm.at[c, pl.ds(s * rpt + k * rpt2, rpt2)])

    return pl.kernel(
        body, out_type=out_type, mesh=_sc_mesh(), scratch_types=scratch,
        compiler_params=pltpu.CompilerParams(use_tc_tiling_on_sc=False),
    )


def _tc0_body(x_ref, wl_ref, wr_ref, bl_ref, p_ref, r_ref):
    x = x_ref[...]
    p_ref[...] = jnp.dot(x, wl_ref[...], preferred_element_type=jnp.float32)
    r_ref[...] = (
        jnp.dot(x, wr_ref[...], preferred_element_type=jnp.float32) + bl_ref[...]
    )


def _bn_relu(out, g, b):
    mu = jnp.mean(out, axis=0, keepdims=True)
    d = out - mu
    var = jnp.mean(d * d, axis=0, keepdims=True)
    return jnp.maximum(d * lax.rsqrt(var + 1e-5) * g + b, 0.0)


def _make_tc_first(n):
    def body(part_ref, cntp_ref, r_ref, g_ref, b_ref, wl_ref, wr_ref, bl_ref,
             cinv_ref, p_ref, r2_ref):
        cnt = cntp_ref[0, :n, 0:1] + cntp_ref[1, :n, 0:1]
        cinv = 1.0 / jnp.maximum(cnt, 1.0)
        cinv_ref[...] = cinv
        agg = (part_ref[0, :n, :] + part_ref[1, :n, :]) * cinv
        h = _bn_relu(agg + r_ref[...], g_ref[...], b_ref[...])
        p_ref[...] = jnp.dot(h, wl_ref[...], preferred_element_type=jnp.float32)
        r2_ref[...] = (
            jnp.dot(h, wr_ref[...], preferred_element_type=jnp.float32) + bl_ref[...]
        )

    return body


def _make_tc_mid(n, project):
    def body(part_ref, cinv_ref, r_ref, g_ref, b_ref, wl_ref, wr_ref, bl_ref,
             p_ref, r2_ref):
        agg = (part_ref[0, :n, :] + part_ref[1, :n, :]) * cinv_ref[...]
        h = _bn_relu(agg + r_ref[...], g_ref[...], b_ref[...])
        if project:
            p_ref[...] = jnp.dot(h, wl_ref[...], preferred_element_type=jnp.float32)
        else:
            p_ref[...] = h
        r2_ref[...] = (
            jnp.dot(h, wr_ref[...], preferred_element_type=jnp.float32) + bl_ref[...]
        )

    return body


def _make_tc_final(n, ng):
    def body(part_ref, cinv_ref, r_ref, wl_ref, batch_ref, out_ref):
        agg = (part_ref[0, :n, :] + part_ref[1, :n, :]) * cinv_ref[...]
        out = (
            jnp.dot(agg, wl_ref[...], preferred_element_type=jnp.float32) + r_ref[...]
        )
        gid = lax.broadcasted_iota(jnp.int32, (1, ng), 1)
        onehot = (batch_ref[...] == gid).astype(jnp.float32)
        s = lax.dot_general(
            onehot, out, (((0,), (0,)), ((), ())),
            preferred_element_type=jnp.float32,
        )
        gc = lax.dot_general(
            onehot, jnp.ones((n, 1), jnp.float32), (((0,), (0,)), ((), ())),
            preferred_element_type=jnp.float32,
        )
        out_ref[...] = s / jnp.maximum(gc, 1.0)

    return body


def kernel(x, edge_index, batch, Wl0, bl0, Wr0, Wl_mid, bl_mid, Wr_mid,
           Wl_last, bl_last, Wr_last, gamma, beta):
    n, d = x.shape
    e = edge_index.shape[1]
    ng = 64
    out_dim = Wl_last.shape[1]

    ch = -(-e // (NW * CB))
    ch = -(-ch // 8) * 8
    e_pad = NW * ch * CB
    n_acc = -(-(n + 1) // (NS * 8)) * (NS * 8)

    src = edge_index[0]
    dst = edge_index[1]
    pad = e_pad - e
    src_p = jnp.concatenate([src, jnp.zeros((pad,), jnp.int32)]).reshape(NW, ch, CB)
    dst_p = jnp.concatenate([dst, jnp.full((pad,), n, jnp.int32)]).reshape(NW, ch, CB)

    sc_first = _make_sc_pass(n, n_acc, ch, with_cnt=True, nb=5)
    sc_rest = _make_sc_pass(n, n_acc, ch, with_cnt=False, nb=10)

    f32 = jnp.float32
    tc0 = pl.pallas_call(
        _tc0_body,
        out_shape=[jax.ShapeDtypeStruct((n, H), f32)] * 2,
    )
    tc_first = pl.pallas_call(
        _make_tc_first(n),
        out_shape=[
            jax.ShapeDtypeStruct((n, 1), f32),
            jax.ShapeDtypeStruct((n, H), f32),
            jax.ShapeDtypeStruct((n, H), f32),
        ],
    )
    tc_mid = pl.pallas_call(
        _make_tc_mid(n, True),
        out_shape=[jax.ShapeDtypeStruct((n, H), f32)] * 2,
    )
    tc_pre_last = pl.pallas_call(
        _make_tc_mid(n, False),
        out_shape=[
            jax.ShapeDtypeStruct((n, H), f32),
            jax.ShapeDtypeStruct((n, out_dim), f32),
        ],
    )
    tc_final = pl.pallas_call(
        _make_tc_final(n, ng),
        out_shape=jax.ShapeDtypeStruct((ng, out_dim), f32),
    )

    g2 = gamma.reshape(5, 1, H)
    b2 = beta.reshape(5, 1, H)

    p, r = tc0(x, Wl0, Wr0, bl0.reshape(1, H))
    part, cntp = sc_first(p, src_p, dst_p)
    cinv, p, r = tc_first(
        part, cntp, r, g2[0], b2[0],
        Wl_mid[0], Wr_mid[0], bl_mid[0].reshape(1, H),
    )
    for i in range(1, 4):
        part, = sc_rest(p, src_p, dst_p)
        p, r = tc_mid(
            part, cinv, r, g2[i], b2[i],
            Wl_mid[i], Wr_mid[i], bl_mid[i].reshape(1, H),
        )
    part, = sc_rest(p, src_p, dst_p)
    p, r = tc_pre_last(
        part, cinv, r, g2[4], b2[4],
        Wl_mid[3], Wr_last, bl_last.reshape(1, out_dim),
    )
    part, = sc_rest(p, src_p, dst_p)
    return tc_final(part, cinv, r, Wl_last, batch.reshape(n, 1))


# speedup vs baseline: 1.1220x; 1.0082x over previous
import jax
import jax.numpy as jnp
from jax import lax
from jax.experimental import pallas as pl
from jax.experimental.pallas import tpu as pltpu
from jax.experimental.pallas import tpu_sc as plsc

NC = 2
NS = 16
NW = NC * NS
CB = 128
H = 32
CNTW = 16


def _sc_mesh():
    return plsc.VectorSubcoreMesh(
        core_axis_name="c", subcore_axis_name="s", num_cores=NC, num_subcores=NS
    )


def _zero_fill(ref, nrows, ncols):
    z = jnp.zeros((16,), jnp.float32)

    def row(i, _):
        for c0 in range(0, ncols, 16):
            ref[i, pl.ds(c0, 16)] = z
        return 0

    lax.fori_loop(0, nrows, row, 0)


def _one_fill(ref, nrows, ncols):
    o = jnp.ones((16,), jnp.float32)

    def row(i, _):
        for c0 in range(0, ncols, 16):
            ref[i, pl.ds(c0, 16)] = o
        return 0

    lax.fori_loop(0, nrows, row, 0)


def _make_sc_pass(n, n_acc, ch, with_cnt, nb):
    rpt = n_acc // NS
    rpt2 = rpt // 2
    spt = n // NS
    assert n % NS == 0 and ch % nb == 0 and rpt % 2 == 0

    out_type = [jax.ShapeDtypeStruct((NC, n_acc, H), jnp.float32)]
    scratch = [
        pltpu.VMEM((ch, CB), jnp.int32),
        pltpu.VMEM((ch, CB), jnp.int32),
        [pltpu.VMEM((CB, H), jnp.float32) for _ in range(nb)],
        pltpu.VMEM((rpt2, H), jnp.float32),
        pltpu.VMEM_SHARED((n, H), jnp.float32),
        pltpu.VMEM_SHARED((n_acc, H), jnp.float32),
        pltpu.SemaphoreType.DMA,
        pltpu.SemaphoreType.DMA,
    ]
    if with_cnt:
        out_type.append(jax.ShapeDtypeStruct((NC, n_acc, CNTW), jnp.float32))
        scratch += [
            pltpu.VMEM((CB, CNTW), jnp.float32),
            pltpu.VMEM((rpt2, CNTW), jnp.float32),
            pltpu.VMEM_SHARED((n_acc, CNTW), jnp.float32),
        ]

    def body(p_hbm, src_hbm, dst_hbm, out_hbm, *rest):
        if with_cnt:
            (cnt_hbm, idx_s, idx_d, rows, zbuf, psh, acc, sem_g, sem_s,
             ones, zcnt, acc_cnt) = rest
        else:
            idx_s, idx_d, rows, zbuf, psh, acc, sem_g, sem_s = rest
        c = lax.axis_index("c")
        s = lax.axis_index("s")
        wid = c * NS + s

        pltpu.async_copy(p_hbm.at[pl.ds(s * spt, spt)],
                         psh.at[pl.ds(s * spt, spt)], sem_g)
        _zero_fill(zbuf, rpt2, H)
        for k in range(2):
            pltpu.sync_copy(zbuf, acc.at[pl.ds(s * rpt + k * rpt2, rpt2)])
        if with_cnt:
            _zero_fill(zcnt, rpt2, CNTW)
            for k in range(2):
                pltpu.sync_copy(
                    zcnt, acc_cnt.at[pl.ds(s * rpt + k * rpt2, rpt2)])
            _one_fill(ones, CB, CNTW)
        pltpu.sync_copy(src_hbm.at[wid], idx_s)
        pltpu.sync_copy(dst_hbm.at[wid], idx_d)
        pltpu.make_async_copy(p_hbm.at[pl.ds(s * spt, spt)],
                              psh.at[pl.ds(s * spt, spt)], sem_g).wait()
        plsc.subcore_barrier()

        def group(g, _):
            j0 = g * nb
            gat = [
                pltpu.async_copy(psh.at[idx_s.at[j0 + b]], rows[b], sem_g)
                for b in range(nb)
            ]
            sca = []
            for b in range(nb):
                gat[b].wait()
                sca.append(pltpu.async_copy(
                    rows[b], acc.at[idx_d.at[j0 + b]], sem_s, add=True))
                if with_cnt:
                    sca.append(pltpu.async_copy(
                        ones, acc_cnt.at[idx_d.at[j0 + b]], sem_s, add=True))
            for d in sca:
                d.wait()
            return 0

        lax.fori_loop(0, ch // nb, group, 0)
        plsc.subcore_barrier()

        for k in range(2):
            pltpu.sync_copy(acc.at[pl.ds(s * rpt + k * rpt2, rpt2)], zbuf)
            pltpu.sync_copy(zbuf, out_hbm.at[c, pl.ds(s * rpt + k * rpt2, rpt2)])
        if with_cnt:
            for k in range(2):
                pltpu.sync_copy(
                    acc_cnt.at[pl.ds(s * rpt + k * rpt2, rpt2)], zcnt)
                pltpu.sync_copy(
                    zcnt, cnt_hbm.at[c, pl.ds(s * rpt + k * rpt2, rpt2)])

    return pl.kernel(
        body, out_type=out_type, mesh=_sc_mesh(), scratch_types=scratch,
        compiler_params=pltpu.CompilerParams(use_tc_tiling_on_sc=False),
    )


def _make_sc_cnt(n_acc, ch, nb):
    rpt = n_acc // NS
    rpt2 = rpt // 2
    assert ch % nb == 0

    scratch = [
        pltpu.VMEM((ch, CB), jnp.int32),
        pltpu.VMEM((CB, CNTW), jnp.float32),
        pltpu.VMEM((rpt2, CNTW), jnp.float32),
        pltpu.VMEM_SHARED((n_acc, CNTW), jnp.float32),
        pltpu.SemaphoreType.DMA,
    ]

    def body(dst_hbm, cnt_hbm, idx_d, ones, zcnt, acc_cnt, sem_s):
        c = lax.axis_index("c")
        s = lax.axis_index("s")
        wid = c * NS + s

        _zero_fill(zcnt, rpt2, CNTW)
        for k in range(2):
            pltpu.sync_copy(zcnt, acc_cnt.at[pl.ds(s * rpt + k * rpt2, rpt2)])
        _one_fill(ones, CB, CNTW)
        pltpu.sync_copy(dst_hbm.at[wid], idx_d)
        plsc.subcore_barrier()

        def group(g, _):
            j0 = g * nb
            sca = [
                pltpu.async_copy(ones, acc_cnt.at[idx_d.at[j0 + b]],
                                 sem_s, add=True)
                for b in range(nb)
            ]
            for d in sca:
                d.wait()
            return 0

        lax.fori_loop(0, ch // nb, group, 0)
        plsc.subcore_barrier()

        for k in range(2):
            pltpu.sync_copy(acc_cnt.at[pl.ds(s * rpt + k * rpt2, rpt2)], zcnt)
            pltpu.sync_copy(zcnt, cnt_hbm.at[c, pl.ds(s * rpt + k * rpt2, rpt2)])

    return pl.kernel(
        body,
        out_type=[jax.ShapeDtypeStruct((NC, n_acc, CNTW), jnp.float32)],
        mesh=_sc_mesh(), scratch_types=scratch,
        compiler_params=pltpu.CompilerParams(use_tc_tiling_on_sc=False),
    )


def _tc0_body(x_ref, wl_ref, wr_ref, bl_ref, p_ref, r_ref):
    x = x_ref[...]
    p_ref[...] = jnp.dot(x, wl_ref[...], preferred_element_type=jnp.float32)
    r_ref[...] = (
        jnp.dot(x, wr_ref[...], preferred_element_type=jnp.float32) + bl_ref[...]
    )


def _bn_relu(out, g, b):
    mu = jnp.mean(out, axis=0, keepdims=True)
    d = out - mu
    var = jnp.mean(d * d, axis=0, keepdims=True)
    return jnp.maximum(d * lax.rsqrt(var + 1e-5) * g + b, 0.0)


def _make_tc_first(n):
    def body(part_ref, cntp_ref, r_ref, g_ref, b_ref, wl_ref, wr_ref, bl_ref,
             cinv_ref, p_ref, r2_ref):
        cnt = cntp_ref[0, :n, 0:1] + cntp_ref[1, :n, 0:1]
        cinv = 1.0 / jnp.maximum(cnt, 1.0)
        cinv_ref[...] = cinv
        agg = (part_ref[0, :n, :] + part_ref[1, :n, :]) * cinv
        h = _bn_relu(agg + r_ref[...], g_ref[...], b_ref[...])
        p_ref[...] = jnp.dot(h, wl_ref[...], preferred_element_type=jnp.float32)
        r2_ref[...] = (
            jnp.dot(h, wr_ref[...], preferred_element_type=jnp.float32) + bl_ref[...]
        )

    return body


def _make_tc_mid(n, project):
    def body(part_ref, cinv_ref, r_ref, g_ref, b_ref, wl_ref, wr_ref, bl_ref,
             p_ref, r2_ref):
        agg = (part_ref[0, :n, :] + part_ref[1, :n, :]) * cinv_ref[...]
        h = _bn_relu(agg + r_ref[...], g_ref[...], b_ref[...])
        if project:
            p_ref[...] = jnp.dot(h, wl_ref[...], preferred_element_type=jnp.float32)
        else:
            p_ref[...] = h
        r2_ref[...] = (
            jnp.dot(h, wr_ref[...], preferred_element_type=jnp.float32) + bl_ref[...]
        )

    return body


def _make_tc_final(n, ng):
    def body(part_ref, cinv_ref, r_ref, wl_ref, batch_ref, out_ref):
        agg = (part_ref[0, :n, :] + part_ref[1, :n, :]) * cinv_ref[...]
        out = (
            jnp.dot(agg, wl_ref[...], preferred_element_type=jnp.float32) + r_ref[...]
        )
        gid = lax.broadcasted_iota(jnp.int32, (1, ng), 1)
        onehot = (batch_ref[...] == gid).astype(jnp.float32)
        s = lax.dot_general(
            onehot, out, (((0,), (0,)), ((), ())),
            preferred_element_type=jnp.float32,
        )
        gc = lax.dot_general(
            onehot, jnp.ones((n, 1), jnp.float32), (((0,), (0,)), ((), ())),
            preferred_element_type=jnp.float32,
        )
        out_ref[...] = s / jnp.maximum(gc, 1.0)

    return body


def kernel(x, edge_index, batch, Wl0, bl0, Wr0, Wl_mid, bl_mid, Wr_mid,
           Wl_last, bl_last, Wr_last, gamma, beta):
    n, d = x.shape
    e = edge_index.shape[1]
    ng = 64
    out_dim = Wl_last.shape[1]

    ch = -(-e // (NW * CB))
    ch = -(-ch // 8) * 8
    e_pad = NW * ch * CB
    n_acc = -(-(n + 1) // (NS * 8)) * (NS * 8)

    src = edge_index[0]
    dst = edge_index[1]
    pad = e_pad - e
    src_p = jnp.concatenate([src, jnp.zeros((pad,), jnp.int32)]).reshape(NW, ch, CB)
    dst_p = jnp.concatenate([dst, jnp.full((pad,), n, jnp.int32)]).reshape(NW, ch, CB)

    sc_cnt = _make_sc_cnt(n_acc, ch, nb=10)
    sc_rest = _make_sc_pass(n, n_acc, ch, with_cnt=False, nb=10)

    f32 = jnp.float32
    tc0 = pl.pallas_call(
        _tc0_body,
        out_shape=[jax.ShapeDtypeStruct((n, H), f32)] * 2,
    )
    tc_first = pl.pallas_call(
        _make_tc_first(n),
        out_shape=[
            jax.ShapeDtypeStruct((n, 1), f32),
            jax.ShapeDtypeStruct((n, H), f32),
            jax.ShapeDtypeStruct((n, H), f32),
        ],
    )
    tc_mid = pl.pallas_call(
        _make_tc_mid(n, True),
        out_shape=[jax.ShapeDtypeStruct((n, H), f32)] * 2,
    )
    tc_pre_last = pl.pallas_call(
        _make_tc_mid(n, False),
        out_shape=[
            jax.ShapeDtypeStruct((n, H), f32),
            jax.ShapeDtypeStruct((n, out_dim), f32),
        ],
    )
    tc_final = pl.pallas_call(
        _make_tc_final(n, ng),
        out_shape=jax.ShapeDtypeStruct((ng, out_dim), f32),
    )

    g2 = gamma.reshape(5, 1, H)
    b2 = beta.reshape(5, 1, H)

    cntp, = sc_cnt(dst_p)
    p, r = tc0(x, Wl0, Wr0, bl0.reshape(1, H))
    part, = sc_rest(p, src_p, dst_p)
    cinv, p, r = tc_first(
        part, cntp, r, g2[0], b2[0],
        Wl_mid[0], Wr_mid[0], bl_mid[0].reshape(1, H),
    )
    for i in range(1, 4):
        part, = sc_rest(p, src_p, dst_p)
        p, r = tc_mid(
            part, cinv, r, g2[i], b2[i],
            Wl_mid[i], Wr_mid[i], bl_mid[i].reshape(1, H),
        )
    part, = sc_rest(p, src_p, dst_p)
    p, r = tc_pre_last(
        part, cinv, r, g2[4], b2[4],
        Wl_mid[3], Wr_last, bl_last.reshape(1, out_dim),
    )
    part, = sc_rest(p, src_p, dst_p)
    return tc_final(part, cinv, r, Wl_last, batch.reshape(n, 1))
